# baseline (speedup 1.0000x reference)
import functools

import jax
import jax.numpy as jnp
from jax import lax
from jax.experimental import pallas as pl
from jax.experimental.pallas import tpu as pltpu
from jax.experimental.pallas import tpu_sc as plsc

BATCH = 16384
DIM = 64
NCLASS = 100000
LANES = 16
NW = 32
ROWS_PER_W = DIM // NW
OCHUNK = 8192
NOCH = BATCH // OCHUNK
GRP = 4
BLK = 2048


def _sc_gather_body(centers_hbm, idx_hbm, out_hbm, row_v, idx_v, gat_v,
                    sem_r):
    wid = lax.axis_index("s") * 2 + lax.axis_index("c")

    def row_body(r, carry):
        d = wid * ROWS_PER_W + r
        cp = pltpu.async_copy(centers_hbm.at[d], row_v, sem_r)

        @pl.when(r == 0)
        def _():
            pltpu.sync_copy(idx_hbm, idx_v)

        cp.wait()

        def chunk_body(c, carry2):
            base = c * OCHUNK

            @plsc.parallel_loop(0, OCHUNK // (LANES * GRP), unroll=2)
            def _(g):
                for t in range(GRP):
                    off = (g * GRP + t) * LANES
                    iv = idx_v[pl.ds(base + off, LANES)]
                    gat_v[pl.ds(off, LANES)] = plsc.load_gather(row_v, [iv])

            pltpu.sync_copy(gat_v, out_hbm.at[d, pl.ds(base, OCHUNK)])
            return carry2

        return lax.fori_loop(0, NOCH, chunk_body, carry)

    lax.fori_loop(0, ROWS_PER_W, row_body, 0)


_sc_gather = functools.partial(
    pl.kernel,
    out_type=jax.ShapeDtypeStruct((DIM, BATCH), jnp.float32),
    mesh=plsc.VectorSubcoreMesh(core_axis_name="c", subcore_axis_name="s"),
    compiler_params=pltpu.CompilerParams(needs_layout_passes=False,
                                         skip_device_barrier=True),
    scratch_types=[
        pltpu.VMEM((NCLASS,), jnp.float32),
        pltpu.VMEM((BATCH,), jnp.int32),
        pltpu.VMEM((OCHUNK,), jnp.float32),
        pltpu.SemaphoreType.DMA,
    ],
)(_sc_gather_body)


def _loss_body(featT_ref, gt_ref, out_ref):
    i = pl.program_id(0)

    @pl.when(i == 0)
    def _():
        out_ref[...] = jnp.zeros_like(out_ref)

    dv = featT_ref[...] - gt_ref[...]
    out_ref[...] = out_ref[...] + jnp.sum(dv * dv)

    @pl.when(i == BATCH // BLK - 1)
    def _():
        out_ref[...] = out_ref[...] * (1.0 / BATCH)


def _tc_loss(featT, gt):
    return pl.pallas_call(
        _loss_body,
        grid=(BATCH // BLK,),
        in_specs=[
            pl.BlockSpec((DIM, BLK), lambda i: (0, i)),
            pl.BlockSpec((DIM, BLK), lambda i: (0, i)),
        ],
        out_specs=pl.BlockSpec((1, 1), lambda i: (0, 0)),
        out_shape=jax.ShapeDtypeStruct((1, 1), jnp.float32),
    )(featT, gt)


def kernel(feat, centers, index):
    idx = index.astype(jnp.int32)
    gt = _sc_gather(centers, idx)
    total = _tc_loss(feat.T, gt)
    return total[0, 0]


# speedup vs baseline: 1.0531x; 1.0531x over previous
import functools

import jax
import jax.numpy as jnp
from jax import lax
from jax.experimental import pallas as pl
from jax.experimental.pallas import tpu as pltpu
from jax.experimental.pallas import tpu_sc as plsc

BATCH = 16384
DIM = 64
NCLASS = 100000
LANES = 16
NW = 32
ROWS_PER_W = DIM // NW
OCHUNK = 8192
NOCH = BATCH // OCHUNK
GRP = 4
BLK = 4096


def _sc_gather_body(centers_hbm, idx_hbm, out_hbm, row_v, idx_v, gat_v,
                    sem_r):
    wid = lax.axis_index("s") * 2 + lax.axis_index("c")

    def row_body(r, carry):
        d = wid * ROWS_PER_W + r
        cp = pltpu.async_copy(centers_hbm.at[d], row_v, sem_r)

        @pl.when(r == 0)
        def _():
            pltpu.sync_copy(idx_hbm, idx_v)

        cp.wait()

        def chunk_body(c, carry2):
            base = c * OCHUNK

            @plsc.parallel_loop(0, OCHUNK // (LANES * GRP), unroll=2)
            def _(g):
                for t in range(GRP):
                    off = (g * GRP + t) * LANES
                    iv = idx_v[pl.ds(base + off, LANES)]
                    gat_v[pl.ds(off, LANES)] = plsc.load_gather(row_v, [iv])

            pltpu.sync_copy(gat_v, out_hbm.at[d, pl.ds(base, OCHUNK)])
            return carry2

        return lax.fori_loop(0, NOCH, chunk_body, carry)

    lax.fori_loop(0, ROWS_PER_W, row_body, 0)


_sc_gather = functools.partial(
    pl.kernel,
    out_type=jax.ShapeDtypeStruct((DIM, BATCH), jnp.float32),
    mesh=plsc.VectorSubcoreMesh(core_axis_name="c", subcore_axis_name="s"),
    compiler_params=pltpu.CompilerParams(needs_layout_passes=False),
    scratch_types=[
        pltpu.VMEM((NCLASS,), jnp.float32),
        pltpu.VMEM((BATCH,), jnp.int32),
        pltpu.VMEM((OCHUNK,), jnp.float32),
        pltpu.SemaphoreType.DMA,
    ],
)(_sc_gather_body)


def _loss_body(featT_ref, gt_ref, out_ref):
    i = pl.program_id(0)

    @pl.when(i == 0)
    def _():
        out_ref[...] = jnp.zeros_like(out_ref)

    dv = featT_ref[...] - gt_ref[...]
    out_ref[...] = out_ref[...] + jnp.sum(dv * dv)

    @pl.when(i == BATCH // BLK - 1)
    def _():
        out_ref[...] = out_ref[...] * (1.0 / BATCH)


def _tc_loss(featT, gt):
    return pl.pallas_call(
        _loss_body,
        grid=(BATCH // BLK,),
        in_specs=[
            pl.BlockSpec((DIM, BLK), lambda i: (0, i)),
            pl.BlockSpec((DIM, BLK), lambda i: (0, i)),
        ],
        out_specs=pl.BlockSpec((1, 1), lambda i: (0, 0)),
        out_shape=jax.ShapeDtypeStruct((1, 1), jnp.float32),
    )(featT, gt)


def kernel(feat, centers, index):
    idx = index.astype(jnp.int32)
    gt = _sc_gather(centers, idx)
    total = _tc_loss(feat.T, gt)
    return total[0, 0]


# speedup vs baseline: 1.0834x; 1.0287x over previous
import functools

import jax
import jax.numpy as jnp
from jax import lax
from jax.experimental import pallas as pl
from jax.experimental.pallas import tpu as pltpu
from jax.experimental.pallas import tpu_sc as plsc

BATCH = 16384
DIM = 64
NCLASS = 100000
LANES = 16
NW = 32
ROWS_PER_W = DIM // NW
FCHUNK = 8192
NFCH = BATCH // FCHUNK
GRP = 4


def _sc_loss_body(centers_hbm, featT_hbm, idx_hbm, out_hbm,
                  row_v, idx_v, feat_v, acc_v, sem_r, sem_f):
    wid = lax.axis_index("s") * 2 + lax.axis_index("c")

    zeros = jnp.zeros((LANES,), jnp.float32)

    def row_body(r, accs):
        d = wid * ROWS_PER_W + r
        cp = pltpu.async_copy(centers_hbm.at[d], row_v, sem_r)

        @pl.when(r == 0)
        def _():
            pltpu.sync_copy(idx_hbm, idx_v)

        cp_f = pltpu.async_copy(featT_hbm.at[d, pl.ds(0, FCHUNK)], feat_v,
                                sem_f)
        cp_f.wait()
        cp.wait()

        def chunk_body(c, accs2):
            base = c * FCHUNK

            @plsc.parallel_loop(0, FCHUNK // (LANES * GRP), unroll=2,
                                carry=accs2)
            def accs3(g, acc_t):
                a = list(acc_t)
                for t in range(GRP):
                    off = (g * GRP + t) * LANES
                    iv = idx_v[pl.ds(base + off, LANES)]
                    fv = feat_v[pl.ds(off, LANES)]
                    gv = plsc.load_gather(row_v, [iv])
                    dv = fv - gv
                    a[t] = a[t] + dv * dv
                return tuple(a)

            @pl.when(c + 1 < NFCH)
            def _():
                pltpu.sync_copy(
                    featT_hbm.at[d, pl.ds((c + 1) * FCHUNK, FCHUNK)], feat_v)

            return accs3

        return lax.fori_loop(0, NFCH, chunk_body, accs)

    accs = lax.fori_loop(0, ROWS_PER_W, row_body,
                         (zeros, zeros, zeros, zeros))
    acc_v[...] = accs[0] + accs[1] + accs[2] + accs[3]
    pltpu.sync_copy(acc_v, out_hbm.at[pl.ds(wid * LANES, LANES)])


_sc_loss = functools.partial(
    pl.kernel,
    out_type=jax.ShapeDtypeStruct((NW * LANES,), jnp.float32),
    mesh=plsc.VectorSubcoreMesh(core_axis_name="c", subcore_axis_name="s"),
    compiler_params=pltpu.CompilerParams(needs_layout_passes=False),
    scratch_types=[
        pltpu.VMEM((NCLASS,), jnp.float32),
        pltpu.VMEM((BATCH,), jnp.int32),
        pltpu.VMEM((FCHUNK,), jnp.float32),
        pltpu.VMEM((LANES,), jnp.float32),
        pltpu.SemaphoreType.DMA,
        pltpu.SemaphoreType.DMA,
    ],
)(_sc_loss_body)


def kernel(feat, centers, index):
    idx = index.astype(jnp.int32)
    partials = _sc_loss(centers, feat.T, idx)
    return jnp.sum(partials) * (1.0 / BATCH)
